# baseline (speedup 1.0000x reference)
import jax
import jax.numpy as jnp
from jax import lax
from jax.experimental import pallas as pl
from jax.experimental.pallas import tpu as pltpu
from jax.experimental.pallas import tpu_sc as plsc

NC, NS = 2, 16
NW = NC * NS
L = 16

B = 16384
D = 32
BPW = B // NW
SPW = 128
NST = BPW // SPW
NG = SPW // L
TW = 8 * D
NT = 1000000 // 8


def _gmf_body(users_hbm, items_hbm, ut_hbm, it_hbm, w_hbm, b_hbm, out_hbm,
              uidx_v, iidx_v, tu_v, ti_v, u_blk, i_blk, w_v, b_v, out_v, sem):
    wid = lax.axis_index("s") * NC + lax.axis_index("c")
    base = wid * BPW

    pltpu.sync_copy(users_hbm.at[pl.ds(base, BPW)], uidx_v)
    pltpu.sync_copy(items_hbm.at[pl.ds(base, BPW)], iidx_v)
    pltpu.sync_copy(w_hbm, w_v)
    pltpu.sync_copy(b_hbm, b_v)

    def tidx_body(k, carry):
        tu = uidx_v[pl.ds(k * L, L)] >> 3
        ti = iidx_v[pl.ds(k * L, L)] >> 3
        tu_v[pl.ds(k * L, L)] = tu
        ti_v[pl.ds(k * L, L)] = ti
        return carry

    lax.fori_loop(0, BPW // L, tidx_body, 0)

    b_vec = b_v[...]
    w_lo = w_v[pl.ds(0, L)]
    w_hi = w_v[pl.ds(L, L)]
    w_s = [w_lo[d] for d in range(L)] + [w_hi[d] for d in range(L)]
    lane = lax.iota(jnp.int32, L)

    for s in range(NST):
        cu = pltpu.async_copy(
            ut_hbm.at[tu_v.at[pl.ds(s * SPW, SPW)]], u_blk, sem)
        ci = pltpu.async_copy(
            it_hbm.at[ti_v.at[pl.ds(s * SPW, SPW)]], i_blk, sem)
        cu.wait()
        ci.wait()

        def group_body(g, carry, s=s):
            idxu = uidx_v[pl.ds(s * SPW + g * L, L)]
            idxi = iidx_v[pl.ds(s * SPW + g * L, L)]
            rem_u = idxu & 7
            rem_i = idxi & 7
            rows = g * L + lane
            acc = jnp.zeros((L,), jnp.float32)
            for d in range(D):
                ug = plsc.load_gather(u_blk, [rows, rem_u + d * 8])
                ig = plsc.load_gather(i_blk, [rows, rem_i + d * 8])
                acc = acc + ug * ig * w_s[d]
            logits = acc + b_vec
            preds = 1.0 / (1.0 + jnp.exp(-logits))
            out_v[pl.ds(s * SPW + g * L, L)] = preds
            return carry

        lax.fori_loop(0, NG, group_body, 0)

    pltpu.sync_copy(out_v, out_hbm.at[pl.ds(base, BPW)])


@jax.jit
def kernel(users, items, user_table, item_table, W, b):
    mesh = plsc.VectorSubcoreMesh(
        core_axis_name="c", subcore_axis_name="s",
        num_cores=NC, num_subcores=NS)
    run = pl.kernel(
        _gmf_body,
        out_type=jax.ShapeDtypeStruct((B,), jnp.float32),
        mesh=mesh,
        scratch_types=[
            pltpu.VMEM((BPW,), jnp.int32),
            pltpu.VMEM((BPW,), jnp.int32),
            pltpu.VMEM((BPW,), jnp.int32),
            pltpu.VMEM((BPW,), jnp.int32),
            pltpu.VMEM((SPW, TW), jnp.float32),
            pltpu.VMEM((SPW, TW), jnp.float32),
            pltpu.VMEM((D,), jnp.float32),
            pltpu.VMEM((L,), jnp.float32),
            pltpu.VMEM((BPW,), jnp.float32),
            pltpu.SemaphoreType.DMA,
        ],
        compiler_params=pltpu.CompilerParams(
            use_tc_tiling_on_sc=False, needs_layout_passes=False),
    )
    ut2 = (user_table.reshape(NT, 8, D).transpose(0, 2, 1).reshape(NT, TW))
    it2 = (item_table.reshape(NT, 8, D).transpose(0, 2, 1).reshape(NT, TW))
    w32 = W.reshape(D).astype(jnp.float32)
    b16 = jnp.broadcast_to(b.astype(jnp.float32), (L,))
    out = run(users.astype(jnp.int32), items.astype(jnp.int32),
              ut2, it2, w32, b16)
    return out.reshape(B, 1)


# speedup vs baseline: 1.5761x; 1.5761x over previous
import jax
import jax.numpy as jnp
from jax import lax
from jax.experimental import pallas as pl
from jax.experimental.pallas import tpu as pltpu
from jax.experimental.pallas import tpu_sc as plsc

NC, NS = 2, 16
NW = NC * NS
L = 16

B = 16384
D = 32
BPW = B // NW
SPW = 256
NST = BPW // SPW
NSTEP = SPW // L
NG = SPW // L


def _gmf_body(users_hbm, items_hbm, ut_hbm, it_hbm, w_hbm, b_hbm, out_hbm,
              uidx_v, iidx_v, u_rows, i_rows, w_v, b_v, out_v, sem):
    wid = lax.axis_index("s") * NC + lax.axis_index("c")
    base = wid * BPW

    pltpu.sync_copy(users_hbm.at[pl.ds(base, BPW)], uidx_v)
    pltpu.sync_copy(items_hbm.at[pl.ds(base, BPW)], iidx_v)
    pltpu.sync_copy(w_hbm, w_v)
    pltpu.sync_copy(b_hbm, b_v)

    b_vec = b_v[...]
    w_lo = w_v[pl.ds(0, L)]
    w_hi = w_v[pl.ds(L, L)]
    w_s = [w_lo[d] for d in range(L)] + [w_hi[d] for d in range(L)]
    lane = lax.iota(jnp.int32, L)
    cols = [jnp.full((L,), d, jnp.int32) for d in range(D)]

    for s in range(NST):
        def fetch_body(k, carry, s=s):
            idxu = uidx_v[pl.ds(s * SPW + k * L, L)]
            idxi = iidx_v[pl.ds(s * SPW + k * L, L)]
            for j in range(L):
                slot = k * L + j
                pltpu.async_copy(ut_hbm.at[idxu[j]], u_rows.at[slot], sem)
                pltpu.async_copy(it_hbm.at[idxi[j]], i_rows.at[slot], sem)
            return carry

        lax.fori_loop(0, NSTEP, fetch_body, 0)

        def drain_body(k, carry):
            pltpu.make_async_copy(ut_hbm.at[0], u_rows.at[0], sem).wait()
            pltpu.make_async_copy(it_hbm.at[0], i_rows.at[0], sem).wait()
            return carry

        lax.fori_loop(0, SPW, drain_body, 0)

        def group_body(g, carry, s=s):
            slots = g * L + lane
            acc = jnp.zeros((L,), jnp.float32)
            for d in range(D):
                ug = plsc.load_gather(u_rows, [slots, cols[d]])
                ig = plsc.load_gather(i_rows, [slots, cols[d]])
                acc = acc + ug * ig * w_s[d]
            logits = acc + b_vec
            preds = 1.0 / (1.0 + jnp.exp(-logits))
            out_v[pl.ds(s * SPW + g * L, L)] = preds
            return carry

        lax.fori_loop(0, NG, group_body, 0)

    pltpu.sync_copy(out_v, out_hbm.at[pl.ds(base, BPW)])


@jax.jit
def kernel(users, items, user_table, item_table, W, b):
    mesh = plsc.VectorSubcoreMesh(
        core_axis_name="c", subcore_axis_name="s",
        num_cores=NC, num_subcores=NS)
    run = pl.kernel(
        _gmf_body,
        out_type=jax.ShapeDtypeStruct((B,), jnp.float32),
        mesh=mesh,
        scratch_types=[
            pltpu.VMEM((BPW,), jnp.int32),
            pltpu.VMEM((BPW,), jnp.int32),
            pltpu.VMEM((SPW, D), jnp.float32),
            pltpu.VMEM((SPW, D), jnp.float32),
            pltpu.VMEM((D,), jnp.float32),
            pltpu.VMEM((L,), jnp.float32),
            pltpu.VMEM((BPW,), jnp.float32),
            pltpu.SemaphoreType.DMA,
        ],
        compiler_params=pltpu.CompilerParams(needs_layout_passes=False),
    )
    w32 = W.reshape(D).astype(jnp.float32)
    b16 = jnp.broadcast_to(b.astype(jnp.float32), (L,))
    out = run(users.astype(jnp.int32), items.astype(jnp.int32),
              user_table, item_table, w32, b16)
    return out.reshape(B, 1)
